# initial kernel scaffold (unmeasured)
import jax
import jax.numpy as jnp
from jax import lax
from jax.experimental import pallas as pl
from jax.experimental.pallas import tpu as pltpu

N_DEV = 8


def kernel(x, router_W, route_idx, expert_W, shared_W):
    n_tok, d_model = x.shape
    n_local = expert_W.shape[0]
    d_hidden = expert_W.shape[2]
    n_experts = router_W.shape[1]

    def body(x_ref, rw_ref, idx_ref, ew_ref, sw_ref, out_ref,
             comm_ref, send_sems, recv_sems):
        my = lax.axis_index("i")
        left = lax.rem(my + N_DEV - 1, N_DEV)
        right = lax.rem(my + 1, N_DEV)

        barrier_sem = pltpu.get_barrier_semaphore()
        for nbr in (left, right):
            pl.semaphore_signal(
                barrier_sem, inc=1,
                device_id=(nbr,), device_id_type=pl.DeviceIdType.MESH,
            )
        pl.semaphore_wait(barrier_sem, 2)

        xv = x_ref[:, :]
        scores = jnp.dot(xv, rw_ref[:, :], preferred_element_type=jnp.float32)
        s_max = jnp.max(scores, axis=1, keepdims=True)
        probs = jnp.exp(scores - s_max)
        probs = probs / jnp.sum(probs, axis=1, keepdims=True)

        idx = idx_ref[:, :]
        lane = lax.broadcasted_iota(jnp.int32, (n_tok, n_experts), 1)

        partial = jnp.zeros((n_tok, d_hidden), jnp.float32)
        for j in range(n_local):
            e = my * n_local + j
            p_e = jnp.sum(jnp.where(lane == e, probs, 0.0),
                          axis=1, keepdims=True)
            coeff = jnp.where(idx == e, p_e, 0.0)
            xj = (xv * coeff).astype(jnp.bfloat16)
            wj = ew_ref[j].astype(jnp.bfloat16)
            partial = partial + jnp.dot(xj, wj,
                                        preferred_element_type=jnp.float32)

        comm_ref[0] = partial.astype(jnp.bfloat16)

        shared = jnp.dot(xv.astype(jnp.bfloat16),
                         sw_ref[:, :].astype(jnp.bfloat16),
                         preferred_element_type=jnp.float32)
        out_ref[:, :] = shared + partial

        for hop in range(N_DEV - 1):
            rdma = pltpu.make_async_remote_copy(
                src_ref=comm_ref.at[hop],
                dst_ref=comm_ref.at[hop + 1],
                send_sem=send_sems.at[hop],
                recv_sem=recv_sems.at[hop],
                device_id=(right,),
                device_id_type=pl.DeviceIdType.MESH,
            )
            rdma.start()
            rdma.wait()
            out_ref[:, :] = out_ref[:, :] + comm_ref[hop + 1].astype(jnp.float32)

    return pl.pallas_call(
        body,
        out_shape=jax.ShapeDtypeStruct((n_tok, d_hidden), jnp.float32),
        in_specs=[pl.BlockSpec(memory_space=pltpu.VMEM)] * 5,
        out_specs=pl.BlockSpec(memory_space=pltpu.VMEM),
        scratch_shapes=[
            pltpu.VMEM((N_DEV, n_tok, d_hidden), jnp.bfloat16),
            pltpu.SemaphoreType.DMA((N_DEV - 1,)),
            pltpu.SemaphoreType.DMA((N_DEV - 1,)),
        ],
        compiler_params=pltpu.CompilerParams(collective_id=0),
    )(x, router_W, route_idx, expert_W, shared_W)


# baseline (device time: 193416 ns/iter reference)
import jax
import jax.numpy as jnp
from jax import lax
from jax.experimental import pallas as pl
from jax.experimental.pallas import tpu as pltpu

N_DEV = 8


def kernel(x, router_W, route_idx, expert_W, shared_W):
    n_tok, d_model = x.shape
    n_local = expert_W.shape[0]
    d_hidden = expert_W.shape[2]
    n_experts = router_W.shape[1]

    def body(x_ref, rw_ref, idx_ref, ew_ref, sw_ref, out_ref,
             comm_ref, send_sems, recv_sems):
        my = lax.axis_index("i")
        left = lax.rem(my + N_DEV - 1, N_DEV)
        right = lax.rem(my + 1, N_DEV)

        barrier_sem = pltpu.get_barrier_semaphore()
        for nbr in (left, right):
            pl.semaphore_signal(
                barrier_sem, inc=1,
                device_id=(nbr,), device_id_type=pl.DeviceIdType.MESH,
            )
        pl.semaphore_wait(barrier_sem, 2)

        xv = x_ref[:, :]
        scores = jnp.dot(xv, rw_ref[:, :], preferred_element_type=jnp.float32)
        s_max = jnp.max(scores, axis=1, keepdims=True)
        probs = jnp.exp(scores - s_max)
        probs = probs / jnp.sum(probs, axis=1, keepdims=True)

        idx = idx_ref[:, :]
        lane = lax.broadcasted_iota(jnp.int32, (n_tok, n_experts), 1)

        comm_ref[0] = jnp.zeros((n_tok, d_hidden), jnp.bfloat16)
        for j in range(n_local):
            e = my * n_local + j
            p_e = jnp.sum(jnp.where(lane == e, probs, 0.0),
                          axis=1, keepdims=True)
            coeff = jnp.where(idx == e, p_e, 0.0)
            xj = (xv * coeff).astype(jnp.bfloat16)
            wj = ew_ref[j].astype(jnp.bfloat16)
            comm_ref[0] = comm_ref[0] + jnp.dot(
                xj, wj, preferred_element_type=jnp.float32
            ).astype(jnp.bfloat16)

        shared = jnp.dot(xv.astype(jnp.bfloat16),
                         sw_ref[:, :].astype(jnp.bfloat16),
                         preferred_element_type=jnp.float32)
        out_ref[:, :] = shared + comm_ref[0].astype(jnp.float32)

        for hop in range(N_DEV - 1):
            rdma = pltpu.make_async_remote_copy(
                src_ref=comm_ref.at[hop],
                dst_ref=comm_ref.at[hop + 1],
                send_sem=send_sems.at[hop],
                recv_sem=recv_sems.at[hop],
                device_id=(right,),
                device_id_type=pl.DeviceIdType.MESH,
            )
            rdma.start()
            rdma.wait()
            out_ref[:, :] = out_ref[:, :] + comm_ref[hop + 1].astype(jnp.float32)

    return pl.pallas_call(
        body,
        out_shape=jax.ShapeDtypeStruct((n_tok, d_hidden), jnp.float32),
        in_specs=[pl.BlockSpec(memory_space=pltpu.VMEM)] * 5,
        out_specs=pl.BlockSpec(memory_space=pltpu.VMEM),
        scratch_shapes=[
            pltpu.VMEM((N_DEV, n_tok, d_hidden), jnp.bfloat16),
            pltpu.SemaphoreType.DMA((N_DEV - 1,)),
            pltpu.SemaphoreType.DMA((N_DEV - 1,)),
        ],
        compiler_params=pltpu.CompilerParams(collective_id=0),
    )(x, router_W, route_idx, expert_W, shared_W)


# device time: 72852 ns/iter; 2.6549x vs baseline; 2.6549x over previous
import jax
import jax.numpy as jnp
from jax import lax
from jax.experimental import pallas as pl
from jax.experimental.pallas import tpu as pltpu

N_DEV = 8
PARTNER_MASKS = (1, 4, 2)


def kernel(x, router_W, route_idx, expert_W, shared_W):
    n_tok, d_model = x.shape
    n_local = expert_W.shape[0]
    d_hidden = expert_W.shape[2]
    n_experts = router_W.shape[1]

    def body(x_ref, rw_ref, idx_ref, ew_ref, sw_ref, out_ref,
             pbuf, gbuf, rbuf0, rbuf1, rbuf2, cbuf,
             rs_send, rs_recv, ag_send, ag_recv):
        p = lax.axis_index("i")

        barrier_sem = pltpu.get_barrier_semaphore()
        for m in PARTNER_MASKS:
            pl.semaphore_signal(
                barrier_sem, inc=1,
                device_id=(p ^ m,), device_id_type=pl.DeviceIdType.MESH,
            )
        pl.semaphore_wait(barrier_sem, len(PARTNER_MASKS))

        xv = x_ref[:, :]
        scores = jnp.dot(xv, rw_ref[:, :], preferred_element_type=jnp.float32)
        s_max = jnp.max(scores, axis=1, keepdims=True)
        probs = jnp.exp(scores - s_max)
        probs = probs / jnp.sum(probs, axis=1, keepdims=True)
        idx = idx_ref[:, :]
        lane = lax.broadcasted_iota(jnp.int32, (n_tok, n_experts), 1)

        for j in range(n_local):
            e = p * n_local + j
            p_e = jnp.sum(jnp.where(lane == e, probs, 0.0),
                          axis=1, keepdims=True)
            cbuf[:, j:j + 1] = jnp.where(idx == e, p_e, 0.0)

        lows = []
        send_los = []
        keep_his = []
        lo = jnp.int32(0)
        sz = n_tok
        for k, m in enumerate(PARTNER_MASKS):
            q = p ^ m
            keep_hi = (p > q).astype(jnp.int32)
            half = sz // 2
            send_lo = lo + (1 - keep_hi) * half
            lo = lo + keep_hi * half
            sz = half
            send_los.append(send_lo)
            keep_his.append(keep_hi)
            lows.append(lo)

        def partial_half(row_lo, rows):
            xh = x_ref[pl.ds(row_lo, rows), :]
            acc = None
            for j in range(n_local):
                cj = cbuf[pl.ds(row_lo, rows), j:j + 1]
                xj = (xh * cj).astype(jnp.bfloat16)
                wj = ew_ref[j].astype(jnp.bfloat16)
                d = jnp.dot(xj, wj, preferred_element_type=jnp.float32)
                acc = d if acc is None else acc + d
            return acc

        h0 = n_tok // 2
        pbuf[pl.ds(send_los[0], h0), :] = partial_half(
            send_los[0], h0).astype(jnp.bfloat16)
        r0 = pltpu.make_async_remote_copy(
            src_ref=pbuf.at[pl.ds(send_los[0], h0), :],
            dst_ref=rbuf0,
            send_sem=rs_send.at[0], recv_sem=rs_recv.at[0],
            device_id=(p ^ PARTNER_MASKS[0],),
            device_id_type=pl.DeviceIdType.MESH,
        )
        r0.start()
        pbuf[pl.ds(lows[0], h0), :] = partial_half(
            lows[0], h0).astype(jnp.bfloat16)
        r0.wait()
        pbuf[pl.ds(lows[0], h0), :] = pbuf[pl.ds(lows[0], h0), :] + rbuf0[:, :]

        h1 = h0 // 2
        r1 = pltpu.make_async_remote_copy(
            src_ref=pbuf.at[pl.ds(send_los[1], h1), :],
            dst_ref=rbuf1,
            send_sem=rs_send.at[1], recv_sem=rs_recv.at[1],
            device_id=(p ^ PARTNER_MASKS[1],),
            device_id_type=pl.DeviceIdType.MESH,
        )
        r1.start()
        seg_lo = lows[2]
        xs = x_ref[pl.ds(seg_lo, n_tok // N_DEV), :]
        shared_seg = jnp.dot(xs.astype(jnp.bfloat16),
                             sw_ref[:, :].astype(jnp.bfloat16),
                             preferred_element_type=jnp.float32)
        r1.wait()
        pbuf[pl.ds(lows[1], h1), :] = pbuf[pl.ds(lows[1], h1), :] + rbuf1[:, :]

        h2 = h1 // 2
        r2 = pltpu.make_async_remote_copy(
            src_ref=pbuf.at[pl.ds(send_los[2], h2), :],
            dst_ref=rbuf2,
            send_sem=rs_send.at[2], recv_sem=rs_recv.at[2],
            device_id=(p ^ PARTNER_MASKS[2],),
            device_id_type=pl.DeviceIdType.MESH,
        )
        r2.start()
        r2.wait()

        seg = (pbuf[pl.ds(seg_lo, h2), :] + rbuf2[:, :]).astype(jnp.float32)
        gbuf[pl.ds(seg_lo, h2), :] = (seg + shared_seg).astype(jnp.bfloat16)

        cur_lo = seg_lo
        cur_sz = h2
        for jj, m in enumerate(reversed(PARTNER_MASKS)):
            rows = cur_sz
            ag = pltpu.make_async_remote_copy(
                src_ref=gbuf.at[pl.ds(cur_lo, rows), :],
                dst_ref=gbuf.at[pl.ds(cur_lo, rows), :],
                send_sem=ag_send.at[jj], recv_sem=ag_recv.at[jj],
                device_id=(p ^ m,),
                device_id_type=pl.DeviceIdType.MESH,
            )
            ag.start()
            ag.wait()
            cur_lo = lows[1 - jj] if jj < 2 else jnp.int32(0)
            cur_sz = cur_sz * 2

        out_ref[:, :] = gbuf[:, :].astype(jnp.float32)

    n_seg = n_tok // N_DEV
    return pl.pallas_call(
        body,
        out_shape=jax.ShapeDtypeStruct((n_tok, d_hidden), jnp.float32),
        in_specs=[pl.BlockSpec(memory_space=pltpu.VMEM)] * 5,
        out_specs=pl.BlockSpec(memory_space=pltpu.VMEM),
        scratch_shapes=[
            pltpu.VMEM((n_tok, d_hidden), jnp.bfloat16),
            pltpu.VMEM((n_tok, d_hidden), jnp.bfloat16),
            pltpu.VMEM((n_tok // 2, d_hidden), jnp.bfloat16),
            pltpu.VMEM((n_tok // 4, d_hidden), jnp.bfloat16),
            pltpu.VMEM((n_seg, d_hidden), jnp.bfloat16),
            pltpu.VMEM((n_tok, n_local), jnp.float32),
            pltpu.SemaphoreType.DMA((3,)),
            pltpu.SemaphoreType.DMA((3,)),
            pltpu.SemaphoreType.DMA((3,)),
            pltpu.SemaphoreType.DMA((3,)),
        ],
        compiler_params=pltpu.CompilerParams(collective_id=0),
    )(x, router_W, route_idx, expert_W, shared_W)


# device time: 71024 ns/iter; 2.7232x vs baseline; 1.0257x over previous
import jax
import jax.numpy as jnp
from jax import lax
from jax.experimental import pallas as pl
from jax.experimental.pallas import tpu as pltpu

N_DEV = 8
PARTNER_MASKS = (3, 1, 4)


def kernel(x, router_W, route_idx, expert_W, shared_W):
    n_tok, d_model = x.shape
    n_local = expert_W.shape[0]
    d_hidden = expert_W.shape[2]
    n_experts = router_W.shape[1]

    def body(x_ref, rw_ref, idx_ref, ew_ref, sw_ref, out_ref,
             pbuf, gbuf, sbuf, rbuf0, rbuf1, rbuf2, cbuf,
             rs_send, rs_recv, ag_send, ag_recv):
        p = lax.axis_index("i")

        barrier_sem = pltpu.get_barrier_semaphore()
        for m in PARTNER_MASKS:
            pl.semaphore_signal(
                barrier_sem, inc=1,
                device_id=(p ^ m,), device_id_type=pl.DeviceIdType.MESH,
            )
        pl.semaphore_wait(barrier_sem, len(PARTNER_MASKS))

        xv = x_ref[:, :]
        scores = jnp.dot(xv, rw_ref[:, :], preferred_element_type=jnp.float32)
        s_max = jnp.max(scores, axis=1, keepdims=True)
        probs = jnp.exp(scores - s_max)
        probs = probs / jnp.sum(probs, axis=1, keepdims=True)
        idx = idx_ref[:, :]
        lane = lax.broadcasted_iota(jnp.int32, (n_tok, n_experts), 1)

        for j in range(n_local):
            e = p * n_local + j
            p_e = jnp.sum(jnp.where(lane == e, probs, 0.0),
                          axis=1, keepdims=True)
            cbuf[:, j:j + 1] = jnp.where(idx == e, p_e, 0.0)

        lows = []
        send_los = []
        lo = jnp.int32(0)
        sz = n_tok
        for m in PARTNER_MASKS:
            q = p ^ m
            keep_hi = (p > q).astype(jnp.int32)
            half = sz // 2
            send_lo = lo + (1 - keep_hi) * half
            lo = lo + keep_hi * half
            sz = half
            send_los.append(send_lo)
            lows.append(lo)

        def partial_half(row_lo, rows):
            xh = x_ref[pl.ds(row_lo, rows), :]
            acc = None
            for j in range(n_local):
                cj = cbuf[pl.ds(row_lo, rows), j:j + 1]
                xj = (xh * cj).astype(jnp.bfloat16)
                wj = ew_ref[j].astype(jnp.bfloat16)
                d = jnp.dot(xj, wj, preferred_element_type=jnp.float32)
                acc = d if acc is None else acc + d
            return acc

        h0 = n_tok // 2
        h1 = h0 // 2
        h2 = h1 // 2

        sbuf[:, :] = partial_half(send_los[0], h0).astype(jnp.bfloat16)
        r0 = pltpu.make_async_remote_copy(
            src_ref=sbuf,
            dst_ref=rbuf0,
            send_sem=rs_send.at[0], recv_sem=rs_recv.at[0],
            device_id=(p ^ PARTNER_MASKS[0],),
            device_id_type=pl.DeviceIdType.MESH,
        )
        r0.start()
        pbuf[pl.ds(lows[0], h0), :] = partial_half(
            lows[0], h0).astype(jnp.bfloat16)
        r0.wait()
        pbuf[pl.ds(lows[0], h0), :] = pbuf[pl.ds(lows[0], h0), :] + rbuf0[:, :]

        r1 = pltpu.make_async_remote_copy(
            src_ref=pbuf.at[pl.ds(send_los[1], h1), :],
            dst_ref=rbuf1,
            send_sem=rs_send.at[1], recv_sem=rs_recv.at[1],
            device_id=(p ^ PARTNER_MASKS[1],),
            device_id_type=pl.DeviceIdType.MESH,
        )
        r1.start()
        seg_lo = lows[2]
        xs = x_ref[pl.ds(seg_lo, h2), :]
        shared_seg = jnp.dot(xs.astype(jnp.bfloat16),
                             sw_ref[:, :].astype(jnp.bfloat16),
                             preferred_element_type=jnp.float32)
        r1.wait()
        pbuf[pl.ds(lows[1], h1), :] = pbuf[pl.ds(lows[1], h1), :] + rbuf1[:, :]

        r2 = pltpu.make_async_remote_copy(
            src_ref=pbuf.at[pl.ds(send_los[2], h2), :],
            dst_ref=rbuf2,
            send_sem=rs_send.at[2], recv_sem=rs_recv.at[2],
            device_id=(p ^ PARTNER_MASKS[2],),
            device_id_type=pl.DeviceIdType.MESH,
        )
        r2.start()
        r2.wait()

        seg = (pbuf[pl.ds(seg_lo, h2), :] + rbuf2[:, :]).astype(jnp.float32)
        gbuf[pl.ds(seg_lo, h2), :] = (seg + shared_seg).astype(jnp.bfloat16)

        cur_lo = seg_lo
        cur_sz = h2
        done_lo = seg_lo
        done_sz = h2
        for jj, m in enumerate(reversed(PARTNER_MASKS)):
            rows = cur_sz
            ag = pltpu.make_async_remote_copy(
                src_ref=gbuf.at[pl.ds(cur_lo, rows), :],
                dst_ref=gbuf.at[pl.ds(cur_lo, rows), :],
                send_sem=ag_send.at[jj], recv_sem=ag_recv.at[jj],
                device_id=(p ^ m,),
                device_id_type=pl.DeviceIdType.MESH,
            )
            ag.start()
            out_ref[pl.ds(done_lo, done_sz), :] = gbuf[
                pl.ds(done_lo, done_sz), :].astype(jnp.float32)
            ag.wait()
            union_lo = lows[1 - jj] if jj < 2 else jnp.int32(0)
            done_lo = jnp.where(cur_lo == union_lo, cur_lo + rows, union_lo)
            done_sz = rows
            cur_lo = union_lo
            cur_sz = cur_sz * 2
        out_ref[pl.ds(done_lo, done_sz), :] = gbuf[
            pl.ds(done_lo, done_sz), :].astype(jnp.float32)

    n_seg = n_tok // N_DEV
    return pl.pallas_call(
        body,
        out_shape=jax.ShapeDtypeStruct((n_tok, d_hidden), jnp.float32),
        in_specs=[pl.BlockSpec(memory_space=pltpu.VMEM)] * 5,
        out_specs=pl.BlockSpec(memory_space=pltpu.VMEM),
        scratch_shapes=[
            pltpu.VMEM((n_tok, d_hidden), jnp.bfloat16),
            pltpu.VMEM((n_tok, d_hidden), jnp.bfloat16),
            pltpu.VMEM((n_tok // 2, d_hidden), jnp.bfloat16),
            pltpu.VMEM((n_tok // 2, d_hidden), jnp.bfloat16),
            pltpu.VMEM((n_tok // 4, d_hidden), jnp.bfloat16),
            pltpu.VMEM((n_seg, d_hidden), jnp.bfloat16),
            pltpu.VMEM((n_tok, n_local), jnp.float32),
            pltpu.SemaphoreType.DMA((3,)),
            pltpu.SemaphoreType.DMA((3,)),
            pltpu.SemaphoreType.DMA((3,)),
            pltpu.SemaphoreType.DMA((3,)),
        ],
        compiler_params=pltpu.CompilerParams(collective_id=0),
    )(x, router_W, route_idx, expert_W, shared_W)


# device time: 50674 ns/iter; 3.8169x vs baseline; 1.4016x over previous
import jax
import jax.numpy as jnp
from jax import lax
from jax.experimental import pallas as pl
from jax.experimental.pallas import tpu as pltpu

N_DEV = 8
MASKS_A = (3, 1, 4)
MASKS_B = (4, 3, 1)


def kernel(x, router_W, route_idx, expert_W, shared_W):
    n_tok, d_model = x.shape
    n_local = expert_W.shape[0]
    d_hidden = expert_W.shape[2]
    n_experts = router_W.shape[1]
    dh = d_hidden // 2

    def body(x_ref, rw_ref, idx_ref, ew_ref, sw_ref, out_ref,
             pbufA, pbufB, gbufA, gbufB, sbufA, sbufB,
             rbufA0, rbufA1, rbufA2, rbufB0, rbufB1, rbufB2, cbuf,
             rs_send, rs_recv, ag_send, ag_recv):
        p = lax.axis_index("i")

        barrier_sem = pltpu.get_barrier_semaphore()
        for m in (1, 3, 4):
            pl.semaphore_signal(
                barrier_sem, inc=1,
                device_id=(p ^ m,), device_id_type=pl.DeviceIdType.MESH,
            )
        pl.semaphore_wait(barrier_sem, 3)

        xv = x_ref[:, :]
        scores = jnp.dot(xv, rw_ref[:, :], preferred_element_type=jnp.float32)
        s_max = jnp.max(scores, axis=1, keepdims=True)
        probs = jnp.exp(scores - s_max)
        probs = probs / jnp.sum(probs, axis=1, keepdims=True)
        idx = idx_ref[:, :]
        lane = lax.broadcasted_iota(jnp.int32, (n_tok, n_experts), 1)

        for j in range(n_local):
            e = p * n_local + j
            p_e = jnp.sum(jnp.where(lane == e, probs, 0.0),
                          axis=1, keepdims=True)
            cbuf[:, j:j + 1] = jnp.where(idx == e, p_e, 0.0)

        def seg_plan(masks):
            lows, send_los = [], []
            lo = jnp.int32(0)
            sz = n_tok
            for m in masks:
                q = p ^ m
                keep_hi = (p > q).astype(jnp.int32)
                half = sz // 2
                send_los.append(lo + (1 - keep_hi) * half)
                lo = lo + keep_hi * half
                sz = half
                lows.append(lo)
            return lows, send_los

        lowsA, sendA = seg_plan(MASKS_A)
        lowsB, sendB = seg_plan(MASKS_B)

        def partial_part(row_lo, rows, col0):
            xh = x_ref[pl.ds(row_lo, rows), :]
            acc = None
            for j in range(n_local):
                cj = cbuf[pl.ds(row_lo, rows), j:j + 1]
                xj = (xh * cj).astype(jnp.bfloat16)
                wj = ew_ref[j, :, col0:col0 + dh].astype(jnp.bfloat16)
                d = jnp.dot(xj, wj, preferred_element_type=jnp.float32)
                acc = d if acc is None else acc + d
            return acc

        def exchange(src, dst, send_sem, recv_sem, mask):
            return pltpu.make_async_remote_copy(
                src_ref=src, dst_ref=dst,
                send_sem=send_sem, recv_sem=recv_sem,
                device_id=(p ^ mask,),
                device_id_type=pl.DeviceIdType.MESH,
            )

        h0 = n_tok // 2
        h1 = h0 // 2
        h2 = h1 // 2

        sbufA[:, :] = partial_part(sendA[0], h0, 0).astype(jnp.bfloat16)
        sbufB[:, :] = partial_part(sendB[0], h0, dh).astype(jnp.bfloat16)
        a0 = exchange(sbufA, rbufA0, rs_send.at[0], rs_recv.at[0], MASKS_A[0])
        b0 = exchange(sbufB, rbufB0, rs_send.at[3], rs_recv.at[3], MASKS_B[0])
        a0.start()
        b0.start()
        pbufA[pl.ds(lowsA[0], h0), :] = partial_part(
            lowsA[0], h0, 0).astype(jnp.bfloat16)
        pbufB[pl.ds(lowsB[0], h0), :] = partial_part(
            lowsB[0], h0, dh).astype(jnp.bfloat16)
        a0.wait()
        b0.wait()
        pbufA[pl.ds(lowsA[0], h0), :] = (
            pbufA[pl.ds(lowsA[0], h0), :] + rbufA0[:, :])
        pbufB[pl.ds(lowsB[0], h0), :] = (
            pbufB[pl.ds(lowsB[0], h0), :] + rbufB0[:, :])

        a1 = exchange(pbufA.at[pl.ds(sendA[1], h1), :], rbufA1,
                      rs_send.at[1], rs_recv.at[1], MASKS_A[1])
        b1 = exchange(pbufB.at[pl.ds(sendB[1], h1), :], rbufB1,
                      rs_send.at[4], rs_recv.at[4], MASKS_B[1])
        a1.start()
        b1.start()
        sw_bf = sw_ref[:, :].astype(jnp.bfloat16)
        sharedA = jnp.dot(
            x_ref[pl.ds(lowsA[2], h2), :].astype(jnp.bfloat16),
            sw_bf[:, 0:dh], preferred_element_type=jnp.float32)
        sharedB = jnp.dot(
            x_ref[pl.ds(lowsB[2], h2), :].astype(jnp.bfloat16),
            sw_bf[:, dh:d_hidden], preferred_element_type=jnp.float32)
        a1.wait()
        b1.wait()
        pbufA[pl.ds(lowsA[1], h1), :] = (
            pbufA[pl.ds(lowsA[1], h1), :] + rbufA1[:, :])
        pbufB[pl.ds(lowsB[1], h1), :] = (
            pbufB[pl.ds(lowsB[1], h1), :] + rbufB1[:, :])

        a2 = exchange(pbufA.at[pl.ds(sendA[2], h2), :], rbufA2,
                      rs_send.at[2], rs_recv.at[2], MASKS_A[2])
        b2 = exchange(pbufB.at[pl.ds(sendB[2], h2), :], rbufB2,
                      rs_send.at[5], rs_recv.at[5], MASKS_B[2])
        a2.start()
        b2.start()
        a2.wait()
        b2.wait()

        segA = (pbufA[pl.ds(lowsA[2], h2), :] + rbufA2[:, :]).astype(
            jnp.float32)
        gbufA[pl.ds(lowsA[2], h2), :] = (segA + sharedA).astype(jnp.bfloat16)
        segB = (pbufB[pl.ds(lowsB[2], h2), :] + rbufB2[:, :]).astype(
            jnp.float32)
        gbufB[pl.ds(lowsB[2], h2), :] = (segB + sharedB).astype(jnp.bfloat16)

        curA, curB = lowsA[2], lowsB[2]
        doneA, doneB = curA, curB
        done_sz = h2
        cur_sz = h2
        for jj in range(3):
            mA = MASKS_A[2 - jj]
            mB = MASKS_B[2 - jj]
            agA = exchange(gbufA.at[pl.ds(curA, cur_sz), :],
                           gbufA.at[pl.ds(curA, cur_sz), :],
                           ag_send.at[jj], ag_recv.at[jj], mA)
            agB = exchange(gbufB.at[pl.ds(curB, cur_sz), :],
                           gbufB.at[pl.ds(curB, cur_sz), :],
                           ag_send.at[3 + jj], ag_recv.at[3 + jj], mB)
            agA.start()
            agB.start()
            out_ref[pl.ds(doneA, done_sz), 0:dh] = gbufA[
                pl.ds(doneA, done_sz), :].astype(jnp.float32)
            out_ref[pl.ds(doneB, done_sz), dh:d_hidden] = gbufB[
                pl.ds(doneB, done_sz), :].astype(jnp.float32)
            agA.wait()
            agB.wait()
            unionA = lowsA[1 - jj] if jj < 2 else jnp.int32(0)
            unionB = lowsB[1 - jj] if jj < 2 else jnp.int32(0)
            doneA = jnp.where(curA == unionA, curA + cur_sz, unionA)
            doneB = jnp.where(curB == unionB, curB + cur_sz, unionB)
            done_sz = cur_sz
            curA, curB = unionA, unionB
            cur_sz = cur_sz * 2
        out_ref[pl.ds(doneA, done_sz), 0:dh] = gbufA[
            pl.ds(doneA, done_sz), :].astype(jnp.float32)
        out_ref[pl.ds(doneB, done_sz), dh:d_hidden] = gbufB[
            pl.ds(doneB, done_sz), :].astype(jnp.float32)

    return pl.pallas_call(
        body,
        out_shape=jax.ShapeDtypeStruct((n_tok, d_hidden), jnp.float32),
        in_specs=[pl.BlockSpec(memory_space=pltpu.VMEM)] * 5,
        out_specs=pl.BlockSpec(memory_space=pltpu.VMEM),
        scratch_shapes=[
            pltpu.VMEM((n_tok, dh), jnp.bfloat16),
            pltpu.VMEM((n_tok, dh), jnp.bfloat16),
            pltpu.VMEM((n_tok, dh), jnp.bfloat16),
            pltpu.VMEM((n_tok, dh), jnp.bfloat16),
            pltpu.VMEM((n_tok // 2, dh), jnp.bfloat16),
            pltpu.VMEM((n_tok // 2, dh), jnp.bfloat16),
            pltpu.VMEM((n_tok // 2, dh), jnp.bfloat16),
            pltpu.VMEM((n_tok // 4, dh), jnp.bfloat16),
            pltpu.VMEM((n_tok // 8, dh), jnp.bfloat16),
            pltpu.VMEM((n_tok // 2, dh), jnp.bfloat16),
            pltpu.VMEM((n_tok // 4, dh), jnp.bfloat16),
            pltpu.VMEM((n_tok // 8, dh), jnp.bfloat16),
            pltpu.VMEM((n_tok, n_local), jnp.float32),
            pltpu.SemaphoreType.DMA((6,)),
            pltpu.SemaphoreType.DMA((6,)),
            pltpu.SemaphoreType.DMA((6,)),
            pltpu.SemaphoreType.DMA((6,)),
        ],
        compiler_params=pltpu.CompilerParams(collective_id=0),
    )(x, router_W, route_idx, expert_W, shared_W)
